# baseline (speedup 1.0000x reference)
import jax
import jax.numpy as jnp
from jax.experimental import pallas as pl
from jax.experimental.pallas import tpu as pltpu

_D = 256


def _copy_body(x_ref, o_ref, sem):
    pltpu.make_async_copy(x_ref, o_ref, sem).start()
    pltpu.make_async_copy(x_ref, o_ref, sem).wait()


def kernel(x):
    x2 = x.reshape(-1, _D)
    m = x2.shape[0]
    return pl.pallas_call(
        _copy_body,
        in_specs=[pl.BlockSpec(memory_space=pl.ANY)],
        out_specs=pl.BlockSpec(memory_space=pl.ANY),
        out_shape=jax.ShapeDtypeStruct((m, _D), x2.dtype),
        scratch_shapes=[pltpu.SemaphoreType.DMA],
    )(x2)


# speedup vs baseline: 30.5143x; 30.5143x over previous
import jax
import jax.numpy as jnp
from jax.experimental import pallas as pl

_D = 256
_BLOCK_ROWS = 1024


def _copy_body(x_ref, o_ref):
    o_ref[...] = x_ref[...]


def kernel(x):
    x2 = x.reshape(-1, _D)
    m = x2.shape[0]
    grid = m // _BLOCK_ROWS
    return pl.pallas_call(
        _copy_body,
        grid=(grid,),
        in_specs=[pl.BlockSpec((_BLOCK_ROWS, _D), lambda i: (i, 0))],
        out_specs=pl.BlockSpec((_BLOCK_ROWS, _D), lambda i: (i, 0)),
        out_shape=jax.ShapeDtypeStruct((m, _D), x2.dtype),
    )(x2)


# speedup vs baseline: 45.3865x; 1.4874x over previous
import jax
import jax.numpy as jnp
from jax.experimental import pallas as pl

_D = 256
_BLOCK_ROWS = 4096


def _copy_body(x_ref, o_ref):
    o_ref[...] = x_ref[...]


def kernel(x):
    x2 = x.reshape(-1, _D)
    m = x2.shape[0]
    grid = m // _BLOCK_ROWS
    return pl.pallas_call(
        _copy_body,
        grid=(grid,),
        in_specs=[pl.BlockSpec((_BLOCK_ROWS, _D), lambda i: (i, 0))],
        out_specs=pl.BlockSpec((_BLOCK_ROWS, _D), lambda i: (i, 0)),
        out_shape=jax.ShapeDtypeStruct((m, _D), x2.dtype),
    )(x2)


# speedup vs baseline: 48.5777x; 1.0703x over previous
import jax
import jax.numpy as jnp
from jax.experimental import pallas as pl

_D = 256
_BLOCK_ROWS = 8192


def _copy_body(x_ref, o_ref):
    o_ref[...] = x_ref[...]


def kernel(x):
    x2 = x.reshape(-1, _D)
    m = x2.shape[0]
    grid = m // _BLOCK_ROWS
    return pl.pallas_call(
        _copy_body,
        grid=(grid,),
        in_specs=[pl.BlockSpec((_BLOCK_ROWS, _D), lambda i: (i, 0))],
        out_specs=pl.BlockSpec((_BLOCK_ROWS, _D), lambda i: (i, 0)),
        out_shape=jax.ShapeDtypeStruct((m, _D), x2.dtype),
    )(x2)
